# baseline (speedup 1.0000x reference)
import functools

import jax
import jax.numpy as jnp
from jax import lax
from jax.experimental import pallas as pl
from jax.experimental.pallas import tpu as pltpu
from jax.experimental.pallas import tpu_sc as plsc

K_TOP = 100
K_SEL = 101
KOUT = 128
QB = 256
CHUNK = 4096
BLK = 128
N_CAND = 100000
N_PAD = 102400
NB = N_PAD // BLK
NB_REAL = 782
D = 32

C1_CAP = 128
S1_CAP = 512

INT_MIN = -2**31


def _score_kernel(q_ref, c_ref, cp_ref, s_ref, bm_ref):
    j = pl.program_id(1)
    q = q_ref[...]
    s = jax.lax.dot_general(q, c_ref[...], (((1,), (1,)), ((), ())),
                            preferred_element_type=jnp.float32)
    s_ref[...] = s
    sp = jax.lax.dot_general(q, cp_ref[...], (((1,), (1,)), ((), ())),
                             preferred_element_type=jnp.float32)
    m = sp[:, 0:32]
    for e in range(1, BLK):
        m = jnp.maximum(m, sp[:, e * 32:(e + 1) * 32])
    for k in range(4):
        @pl.when(j % 4 == k)
        def _():
            bm_ref[:, k * 32:(k + 1) * 32] = m


def _scores_and_blockmax(q, cand_pad, cand_perm):
    nq = q.shape[0]
    grid = (nq // QB, N_PAD // CHUNK)
    return pl.pallas_call(
        _score_kernel,
        grid=grid,
        in_specs=[
            pl.BlockSpec((QB, D), lambda i, j: (i, 0)),
            pl.BlockSpec((CHUNK, D), lambda i, j: (j, 0)),
            pl.BlockSpec((CHUNK, D), lambda i, j: (j, 0)),
        ],
        out_specs=[
            pl.BlockSpec((QB, CHUNK), lambda i, j: (i, j)),
            pl.BlockSpec((QB, 128), lambda i, j: (i, j // 4)),
        ],
        out_shape=[
            jax.ShapeDtypeStruct((nq, N_PAD), jnp.float32),
            jax.ShapeDtypeStruct((nq, NB), jnp.float32),
        ],
    )(q, cand_pad, cand_perm)


def _f32_key(v):
    s = plsc.bitcast(v, jnp.int32)
    return s ^ (jnp.right_shift(s, 31) & jnp.int32(0x7FFFFFFF))


def _ubits(k, shift, mask):
    kx = k ^ jnp.int32(INT_MIN)
    return lax.shift_right_logical(kx, jnp.int32(shift)) & jnp.int32(mask)


def _make_sc_topk(nq):
    info = plsc.get_sparse_core_info()
    nw = info.num_cores * info.num_subcores
    qpw = nq // nw
    mesh = plsc.VectorSubcoreMesh(core_axis_name="c", subcore_axis_name="s")

    n_bm_vecs = NB // 16

    @functools.partial(
        pl.kernel,
        out_type=[
            jax.ShapeDtypeStruct((nq * KOUT,), jnp.float32),
            jax.ShapeDtypeStruct((nq * KOUT,), jnp.int32),
        ],
        mesh=mesh,
        compiler_params=pltpu.CompilerParams(needs_layout_passes=False),
        scratch_types=[
            pltpu.VMEM((NB,), jnp.float32),
            pltpu.VMEM((NB,), jnp.int32),
            pltpu.VMEM((64 * 16,), jnp.int32),
            pltpu.VMEM((256 * 16,), jnp.int32),
            pltpu.VMEM((256 * 16,), jnp.int32),
            pltpu.VMEM((C1_CAP,), jnp.int32),
            pltpu.VMEM((C1_CAP, BLK), jnp.float32),
            pltpu.VMEM((S1_CAP,), jnp.int32),
            pltpu.VMEM((S1_CAP,), jnp.int32),
            pltpu.VMEM((KOUT,), jnp.float32),
            pltpu.VMEM((KOUT,), jnp.int32),
            pltpu.SemaphoreType.DMA,
        ],
    )
    def sc_topk(scores_hbm, bm_hbm, out_s_hbm, out_i_hbm,
                bm_v, key_v, histA, histB, histC, blkids, rows_v,
                s1k, s1i, outs_v, outi_v, sem):
        wid = lax.axis_index("s") * info.num_cores + lax.axis_index("c")
        iota = lax.iota(jnp.int32, 16)
        ones = jnp.ones((16,), jnp.int32)
        zeros_i = jnp.zeros((16,), jnp.int32)

        def per_query(qq, _):
            q = wid * qpw + qq

            pltpu.sync_copy(bm_hbm.at[pl.ds(q * NB, NB)], bm_v)

            def zero_hist(v, _):
                idx = v * 16 + iota
                plsc.store_scatter(histA, [idx & jnp.int32(1023)], zeros_i,
                                   mask=jnp.broadcast_to(v < 64, (16,)))
                plsc.store_scatter(histB, [idx & jnp.int32(4095)], zeros_i)
                plsc.store_scatter(histC, [idx & jnp.int32(4095)], zeros_i)
                plsc.store_scatter(blkids, [idx & jnp.int32(C1_CAP - 1)],
                                   zeros_i,
                                   mask=jnp.broadcast_to(
                                       v < (C1_CAP // 16), (16,)))
                return 0
            lax.fori_loop(0, 256, zero_hist, 0)

            def passA(v, _):
                bid = v * 16 + iota
                valid = bid < NB_REAL
                bm16 = plsc.load_gather(bm_v, [bid])
                k = _f32_key(bm16)
                plsc.store_scatter(key_v, [bid], k)
                binA = _ubits(k, 26, 63)
                plsc.addupdate_scatter(histA, [binA * 16 + iota], ones,
                                       mask=valid)
                return 0
            lax.fori_loop(0, n_bm_vecs, passA, 0)

            def rev_scan(hist, nbins, start):
                def cond(carry):
                    return carry[0] < K_SEL
                def body(carry):
                    acc, b, _ = carry
                    cnt = jnp.sum(plsc.load_gather(hist, [b * 16 + iota]))
                    return (acc + cnt, b - 1, acc)
                accf, bf, above = lax.while_loop(
                    cond, body, (start, jnp.int32(nbins - 1), start))
                return bf + 1, above
            a_star, a_above = rev_scan(histA, 64, 0)

            def passB(v, _):
                bid = v * 16 + iota
                valid = bid < NB_REAL
                k = plsc.load_gather(key_v, [bid])
                binA = _ubits(k, 26, 63)
                binB = _ubits(k, 18, 255)
                m = valid & (binA == a_star)
                plsc.addupdate_scatter(histB, [binB * 16 + iota], ones, mask=m)
                return 0
            lax.fori_loop(0, n_bm_vecs, passB, 0)

            b_star, b_above = rev_scan(histB, 256, a_above)

            def passC(v, _):
                bid = v * 16 + iota
                valid = bid < NB_REAL
                k = plsc.load_gather(key_v, [bid])
                binA = _ubits(k, 26, 63)
                binB = _ubits(k, 18, 255)
                binC = _ubits(k, 10, 255)
                m = valid & (binA == a_star) & (binB == b_star)
                plsc.addupdate_scatter(histC, [binC * 16 + iota], ones, mask=m)
                return 0
            lax.fori_loop(0, n_bm_vecs, passC, 0)

            c_star, _ = rev_scan(histC, 256, b_above)

            kx_t1 = ((a_star << jnp.int32(26)) | (b_star << jnp.int32(18))
                     | (c_star << jnp.int32(10)))
            t1 = kx_t1 ^ jnp.int32(INT_MIN)

            def compact_blocks(v, base):
                bid = v * 16 + iota
                valid = bid < NB_REAL
                k = plsc.load_gather(key_v, [bid])
                m = valid & (k >= t1)
                mi = jnp.where(m, 1, 0)
                pos = base - 1 + plsc.cumsum(mi)
                pm = m & (pos < C1_CAP)
                plsc.store_scatter(blkids, [pos & jnp.int32(C1_CAP - 1)],
                                   bid + q * NB, mask=pm)
                return base + jnp.sum(mi)
            c1 = lax.fori_loop(0, n_bm_vecs, compact_blocks, 0)
            c1 = jnp.minimum(c1, C1_CAP)

            cp1 = pltpu.async_copy(scores_hbm.at[blkids], rows_v, sem)
            cp1.wait()

            def filt(r, base):
                b_r = plsc.load_gather(blkids, [(iota * 0) + r]) - q * NB

                for h in range(BLK // 16):
                    col = h * 16 + iota
                    vals = plsc.load_gather(rows_v, [(iota * 0) + r, col])
                    k = _f32_key(vals)
                    gidx = b_r * BLK + col
                    m = (k >= t1) & (gidx < N_CAND)
                    mi = jnp.where(m, 1, 0)
                    pos = base - 1 + plsc.cumsum(mi)
                    pm = m & (pos < S1_CAP)
                    plsc.store_scatter(s1k, [pos & jnp.int32(S1_CAP - 1)],
                                       k, mask=pm)
                    plsc.store_scatter(s1i, [pos & jnp.int32(S1_CAP - 1)],
                                       gidx, mask=pm)
                    base = base + jnp.sum(mi)
                return base
            s1n = lax.fori_loop(0, c1, filt, 0)
            s1n = jnp.minimum(s1n, S1_CAP)

            def zero_out(v, _):
                idx = v * 16 + iota
                m = idx < KOUT
                plsc.store_scatter(outs_v, [idx & jnp.int32(KOUT - 1)],
                                   jnp.zeros((16,), jnp.float32), mask=m)
                plsc.store_scatter(outi_v, [idx & jnp.int32(KOUT - 1)],
                                   zeros_i, mask=m)
                return 0
            lax.fori_loop(0, (KOUT + 15) // 16, zero_out, 0)

            nv = (s1n + 15) >> 4
            rot_idx = [((iota + r) & jnp.int32(15)) for r in range(16)]

            def rank_a(a, _):
                lane_a = a * 16 + iota
                ka = plsc.load_gather(s1k, [lane_a & jnp.int32(S1_CAP - 1)])
                va = lane_a < s1n

                def rank_b(b, rank):
                    lane_b = b * 16 + iota
                    kb = plsc.load_gather(
                        s1k, [lane_b & jnp.int32(S1_CAP - 1)])
                    kbm = jnp.where(lane_b < s1n, kb, jnp.int32(INT_MIN))
                    for r in range(16):
                        rot = kbm.at[rot_idx[r]].get(
                            mode="promise_in_bounds")
                        rot_pos = b * 16 + rot_idx[r]
                        beats = (rot > ka) | ((rot == ka) &
                                              (rot_pos < lane_a) &
                                              (rot != jnp.int32(INT_MIN)))
                        rank = rank + jnp.where(beats, 1, 0)
                    return rank
                rank = lax.fori_loop(0, nv, rank_b, zeros_i)

                ia = plsc.load_gather(s1i, [lane_a & jnp.int32(S1_CAP - 1)])
                fa = plsc.bitcast(
                    ka ^ (jnp.right_shift(ka, 31) & jnp.int32(0x7FFFFFFF)),
                    jnp.float32)
                m = va & (rank < K_TOP)
                plsc.store_scatter(outs_v, [rank & jnp.int32(127)], fa,
                                   mask=m)
                plsc.store_scatter(outi_v, [rank & jnp.int32(127)], ia,
                                   mask=m)
                return 0
            lax.fori_loop(0, nv, rank_a, 0)

            dbg = jnp.where(iota == 0, c1,
                  jnp.where(iota == 1, s1n,
                  jnp.where(iota == 2, t1,
                  jnp.where(iota == 3, a_star,
                  jnp.where(iota == 4, b_star,
                  jnp.where(iota == 5, c_star,
                  jnp.where(iota == 6, a_above, b_above)))))))
            plsc.store_scatter(outi_v, [iota + jnp.int32(K_TOP)], dbg)

            pltpu.sync_copy(outs_v, out_s_hbm.at[pl.ds(q * KOUT, KOUT)])
            pltpu.sync_copy(outi_v, out_i_hbm.at[pl.ds(q * KOUT, KOUT)])
            return 0

        lax.fori_loop(0, qpw, per_query, 0)

    return sc_topk


@jax.jit
def kernel(query_embeddings, candidates):
    n = candidates.shape[0]
    cand = jnp.pad(candidates, ((0, N_PAD - n), (0, 0)))
    nq = query_embeddings.shape[0]
    cand_perm = (cand.reshape(N_PAD // CHUNK, 32, BLK, D)
                 .transpose(0, 2, 1, 3).reshape(N_PAD, D))
    scores, blockmax = _scores_and_blockmax(query_embeddings, cand, cand_perm)
    scores3 = scores.reshape(nq * NB, BLK)
    bm_flat = blockmax.reshape(nq * NB)
    out_s, out_i = _make_sc_topk(nq)(scores3, bm_flat)
    return (out_s.reshape(nq, KOUT)[:, :K_TOP],
            out_i.reshape(nq, KOUT)[:, :K_TOP])


# speedup vs baseline: 1.0717x; 1.0717x over previous
import functools

import jax
import jax.numpy as jnp
from jax import lax
from jax.experimental import pallas as pl
from jax.experimental.pallas import tpu as pltpu
from jax.experimental.pallas import tpu_sc as plsc

K_TOP = 100
K_SEL = 101
KOUT = 128
QB = 256
CHUNK = 4096
BLK = 128
N_CAND = 100000
N_PAD = 102400
NB = N_PAD // BLK
NB_REAL = 782
D = 32

C1_CAP = 128
S1_CAP = 512

INT_MIN = -2**31


def _score_kernel(q_ref, c_ref, cp_ref, s_ref, bm_ref):
    j = pl.program_id(1)
    q = q_ref[...]
    s = jax.lax.dot_general(q, c_ref[...], (((1,), (1,)), ((), ())),
                            preferred_element_type=jnp.float32)
    s_ref[...] = s
    sp = jax.lax.dot_general(q, cp_ref[...], (((1,), (1,)), ((), ())),
                             preferred_element_type=jnp.float32)
    m = sp[:, 0:32]
    for e in range(1, BLK):
        m = jnp.maximum(m, sp[:, e * 32:(e + 1) * 32])
    for k in range(4):
        @pl.when(j % 4 == k)
        def _():
            bm_ref[:, k * 32:(k + 1) * 32] = m


def _scores_and_blockmax(q, cand_pad, cand_perm):
    nq = q.shape[0]
    grid = (nq // QB, N_PAD // CHUNK)
    return pl.pallas_call(
        _score_kernel,
        grid=grid,
        in_specs=[
            pl.BlockSpec((QB, D), lambda i, j: (i, 0)),
            pl.BlockSpec((CHUNK, D), lambda i, j: (j, 0)),
            pl.BlockSpec((CHUNK, D), lambda i, j: (j, 0)),
        ],
        out_specs=[
            pl.BlockSpec((QB, CHUNK), lambda i, j: (i, j)),
            pl.BlockSpec((QB, 128), lambda i, j: (i, j // 4)),
        ],
        out_shape=[
            jax.ShapeDtypeStruct((nq, N_PAD), jnp.float32),
            jax.ShapeDtypeStruct((nq, NB), jnp.float32),
        ],
    )(q, cand_pad, cand_perm)


def _f32_key(v):
    s = plsc.bitcast(v, jnp.int32)
    return s ^ (jnp.right_shift(s, 31) & jnp.int32(0x7FFFFFFF))


def _ubits(k, shift, mask):
    kx = k ^ jnp.int32(INT_MIN)
    return lax.shift_right_logical(kx, jnp.int32(shift)) & jnp.int32(mask)


def _make_sc_topk(nq):
    info = plsc.get_sparse_core_info()
    nw = info.num_cores * info.num_subcores
    qpw = nq // nw
    mesh = plsc.VectorSubcoreMesh(core_axis_name="c", subcore_axis_name="s")

    n_bm_vecs = NB // 16

    @functools.partial(
        pl.kernel,
        out_type=[
            jax.ShapeDtypeStruct((nq * KOUT,), jnp.float32),
            jax.ShapeDtypeStruct((nq * KOUT,), jnp.int32),
        ],
        mesh=mesh,
        compiler_params=pltpu.CompilerParams(needs_layout_passes=False),
        scratch_types=[
            pltpu.VMEM((NB,), jnp.float32),
            pltpu.VMEM((NB,), jnp.int32),
            pltpu.VMEM((64 * 16,), jnp.int32),
            pltpu.VMEM((256 * 16,), jnp.int32),
            pltpu.VMEM((256 * 16,), jnp.int32),
            pltpu.VMEM((C1_CAP,), jnp.int32),
            pltpu.VMEM((C1_CAP, BLK), jnp.float32),
            pltpu.VMEM((S1_CAP,), jnp.int32),
            pltpu.VMEM((S1_CAP,), jnp.int32),
            pltpu.VMEM((KOUT,), jnp.float32),
            pltpu.VMEM((KOUT,), jnp.int32),
            pltpu.SemaphoreType.DMA,
        ],
    )
    def sc_topk(scores_hbm, bm_hbm, out_s_hbm, out_i_hbm,
                bm_v, key_v, histA, histB, histC, blkids, rows_v,
                s1k, s1i, outs_v, outi_v, sem):
        wid = lax.axis_index("s") * info.num_cores + lax.axis_index("c")
        iota = lax.iota(jnp.int32, 16)
        ones = jnp.ones((16,), jnp.int32)
        zeros_i = jnp.zeros((16,), jnp.int32)

        def per_query(qq, _):
            q = wid * qpw + qq

            pltpu.sync_copy(bm_hbm.at[pl.ds(q * NB, NB)], bm_v)

            def zero_hist(v, _):
                idx = v * 16 + iota
                plsc.store_scatter(histA, [idx & jnp.int32(1023)], zeros_i,
                                   mask=jnp.broadcast_to(v < 64, (16,)))
                plsc.store_scatter(histB, [idx & jnp.int32(4095)], zeros_i)
                plsc.store_scatter(histC, [idx & jnp.int32(4095)], zeros_i)
                plsc.store_scatter(blkids, [idx & jnp.int32(C1_CAP - 1)],
                                   zeros_i,
                                   mask=jnp.broadcast_to(
                                       v < (C1_CAP // 16), (16,)))
                return 0
            lax.fori_loop(0, 256, zero_hist, 0)

            def passA(v, _):
                bid = v * 16 + iota
                valid = bid < NB_REAL
                bm16 = plsc.load_gather(bm_v, [bid])
                k = _f32_key(bm16)
                plsc.store_scatter(key_v, [bid], k)
                binA = _ubits(k, 26, 63)
                plsc.addupdate_scatter(histA, [binA * 16 + iota], ones,
                                       mask=valid)
                return 0
            lax.fori_loop(0, n_bm_vecs, passA, 0)

            def revA(i, carry):
                acc, a_star, a_above = carry
                b = 63 - i
                cnt = jnp.sum(plsc.load_gather(histA, [b * 16 + iota]))
                nacc = acc + cnt
                crossed = (acc < K_SEL) & (nacc >= K_SEL)
                return (nacc,
                        jnp.where(crossed, b, a_star),
                        jnp.where(crossed, acc, a_above))
            _, a_star, a_above = lax.fori_loop(0, 64, revA, (0, 0, 0))

            def passB(v, _):
                bid = v * 16 + iota
                valid = bid < NB_REAL
                k = plsc.load_gather(key_v, [bid])
                binA = _ubits(k, 26, 63)
                binB = _ubits(k, 18, 255)
                m = valid & (binA == a_star)
                plsc.addupdate_scatter(histB, [binB * 16 + iota], ones, mask=m)
                return 0
            lax.fori_loop(0, n_bm_vecs, passB, 0)

            def revB(i, carry):
                acc, b_star, b_above = carry
                b = 255 - i
                cnt = jnp.sum(plsc.load_gather(histB, [b * 16 + iota]))
                nacc = acc + cnt
                crossed = (acc < K_SEL) & (nacc >= K_SEL)
                return (nacc,
                        jnp.where(crossed, b, b_star),
                        jnp.where(crossed, acc, b_above))
            _, b_star, b_above = lax.fori_loop(
                0, 256, revB, (a_above, 0, a_above))

            def passC(v, _):
                bid = v * 16 + iota
                valid = bid < NB_REAL
                k = plsc.load_gather(key_v, [bid])
                binA = _ubits(k, 26, 63)
                binB = _ubits(k, 18, 255)
                binC = _ubits(k, 10, 255)
                m = valid & (binA == a_star) & (binB == b_star)
                plsc.addupdate_scatter(histC, [binC * 16 + iota], ones, mask=m)
                return 0
            lax.fori_loop(0, n_bm_vecs, passC, 0)

            def revC(i, carry):
                acc, c_star = carry
                b = 255 - i
                cnt = jnp.sum(plsc.load_gather(histC, [b * 16 + iota]))
                nacc = acc + cnt
                crossed = (acc < K_SEL) & (nacc >= K_SEL)
                return (nacc, jnp.where(crossed, b, c_star))
            _, c_star = lax.fori_loop(0, 256, revC, (b_above, 0))

            kx_t1 = ((a_star << jnp.int32(26)) | (b_star << jnp.int32(18))
                     | (c_star << jnp.int32(10)))
            t1 = kx_t1 ^ jnp.int32(INT_MIN)

            def compact_blocks(v, base):
                bid = v * 16 + iota
                valid = bid < NB_REAL
                k = plsc.load_gather(key_v, [bid])
                m = valid & (k >= t1)
                mi = jnp.where(m, 1, 0)
                pos = base - 1 + plsc.cumsum(mi)
                pm = m & (pos < C1_CAP)
                plsc.store_scatter(blkids, [pos & jnp.int32(C1_CAP - 1)],
                                   bid + q * NB, mask=pm)
                return base + jnp.sum(mi)
            c1 = lax.fori_loop(0, n_bm_vecs, compact_blocks, 0)
            c1 = jnp.minimum(c1, C1_CAP)

            cp1 = pltpu.async_copy(scores_hbm.at[blkids], rows_v, sem)
            cp1.wait()

            def filt(r, base):
                b_r = plsc.load_gather(blkids, [(iota * 0) + r]) - q * NB

                for h in range(BLK // 16):
                    col = h * 16 + iota
                    vals = plsc.load_gather(rows_v, [(iota * 0) + r, col])
                    k = _f32_key(vals)
                    gidx = b_r * BLK + col
                    m = (k >= t1) & (gidx < N_CAND)
                    mi = jnp.where(m, 1, 0)
                    pos = base - 1 + plsc.cumsum(mi)
                    pm = m & (pos < S1_CAP)
                    plsc.store_scatter(s1k, [pos & jnp.int32(S1_CAP - 1)],
                                       k, mask=pm)
                    plsc.store_scatter(s1i, [pos & jnp.int32(S1_CAP - 1)],
                                       gidx, mask=pm)
                    base = base + jnp.sum(mi)
                return base
            s1n = lax.fori_loop(0, c1, filt, 0)
            s1n = jnp.minimum(s1n, S1_CAP)

            def zero_out(v, _):
                idx = v * 16 + iota
                m = idx < KOUT
                plsc.store_scatter(outs_v, [idx & jnp.int32(KOUT - 1)],
                                   jnp.zeros((16,), jnp.float32), mask=m)
                plsc.store_scatter(outi_v, [idx & jnp.int32(KOUT - 1)],
                                   zeros_i, mask=m)
                return 0
            lax.fori_loop(0, (KOUT + 15) // 16, zero_out, 0)

            nv = (s1n + 15) >> 4
            rot_idx = [((iota + r) & jnp.int32(15)) for r in range(16)]

            def rank_a(a, _):
                lane_a = a * 16 + iota
                ka = plsc.load_gather(s1k, [lane_a & jnp.int32(S1_CAP - 1)])
                va = lane_a < s1n

                def rank_b(b, rank):
                    lane_b = b * 16 + iota
                    kb = plsc.load_gather(
                        s1k, [lane_b & jnp.int32(S1_CAP - 1)])
                    kbm = jnp.where(lane_b < s1n, kb, jnp.int32(INT_MIN))
                    for r in range(16):
                        rot = kbm.at[rot_idx[r]].get(
                            mode="promise_in_bounds")
                        rot_pos = b * 16 + rot_idx[r]
                        beats = (rot > ka) | ((rot == ka) &
                                              (rot_pos < lane_a) &
                                              (rot != jnp.int32(INT_MIN)))
                        rank = rank + jnp.where(beats, 1, 0)
                    return rank
                rank = lax.fori_loop(0, nv, rank_b, zeros_i)

                ia = plsc.load_gather(s1i, [lane_a & jnp.int32(S1_CAP - 1)])
                fa = plsc.bitcast(
                    ka ^ (jnp.right_shift(ka, 31) & jnp.int32(0x7FFFFFFF)),
                    jnp.float32)
                m = va & (rank < K_TOP)
                plsc.store_scatter(outs_v, [rank & jnp.int32(127)], fa,
                                   mask=m)
                plsc.store_scatter(outi_v, [rank & jnp.int32(127)], ia,
                                   mask=m)
                return 0
            lax.fori_loop(0, nv, rank_a, 0)

            dbg = jnp.where(iota == 0, c1,
                  jnp.where(iota == 1, s1n,
                  jnp.where(iota == 2, t1,
                  jnp.where(iota == 3, a_star,
                  jnp.where(iota == 4, b_star,
                  jnp.where(iota == 5, c_star,
                  jnp.where(iota == 6, a_above, b_above)))))))
            plsc.store_scatter(outi_v, [iota + jnp.int32(K_TOP)], dbg)

            pltpu.sync_copy(outs_v, out_s_hbm.at[pl.ds(q * KOUT, KOUT)])
            pltpu.sync_copy(outi_v, out_i_hbm.at[pl.ds(q * KOUT, KOUT)])
            return 0

        lax.fori_loop(0, qpw, per_query, 0)

    return sc_topk


@jax.jit
def kernel(query_embeddings, candidates):
    n = candidates.shape[0]
    cand = jnp.pad(candidates, ((0, N_PAD - n), (0, 0)))
    nq = query_embeddings.shape[0]
    cand_perm = (cand.reshape(N_PAD // CHUNK, 32, BLK, D)
                 .transpose(0, 2, 1, 3).reshape(N_PAD, D))
    scores, blockmax = _scores_and_blockmax(query_embeddings, cand, cand_perm)
    scores3 = scores.reshape(nq * NB, BLK)
    bm_flat = blockmax.reshape(nq * NB)
    out_s, out_i = _make_sc_topk(nq)(scores3, bm_flat)
    return (out_s.reshape(nq, KOUT)[:, :K_TOP],
            out_i.reshape(nq, KOUT)[:, :K_TOP])
